# baseline (speedup 1.0000x reference)
import functools

import jax
import jax.numpy as jnp
from jax import lax
from jax.experimental import pallas as pl
from jax.experimental.pallas import tpu as pltpu
from jax.experimental.pallas import tpu_sc as plsc

D = 64
CH = 128
NC = 2
NS = 16
NW = NC * NS


@functools.partial(jax.jit, static_argnums=(3,))
def _gather(pid3, half3, table128, n_rows):
    n_per_w = n_rows // NW
    n_chunks = n_per_w // CH

    @functools.partial(
        pl.kernel,
        out_type=jax.ShapeDtypeStruct((n_rows, D), jnp.float32),
        mesh=plsc.VectorSubcoreMesh(core_axis_name="c", subcore_axis_name="s"),
        compiler_params=pltpu.CompilerParams(
            use_tc_tiling_on_sc=False, needs_layout_passes=False),
        scratch_types=[
            pltpu.VMEM((n_chunks, CH), jnp.int32),
            pltpu.VMEM((n_chunks, CH), jnp.int32),
            pltpu.VMEM((CH, 2 * D), jnp.float32),
            pltpu.VMEM((CH, 2 * D), jnp.float32),
            pltpu.VMEM((CH, D), jnp.float32),
            pltpu.VMEM((CH, D), jnp.float32),
            pltpu.SemaphoreType.DMA,
            pltpu.SemaphoreType.DMA,
            pltpu.SemaphoreType.DMA,
            pltpu.SemaphoreType.DMA,
        ],
    )
    def k(pid_hbm, half_hbm, table_hbm, out_hbm, pid_v, half_v,
          pbuf_a, pbuf_b, obuf_a, obuf_b, gsem_a, gsem_b, ssem_a, ssem_b):
        wid = lax.axis_index("s") * NC + lax.axis_index("c")
        base = wid * n_per_w
        pltpu.sync_copy(pid_hbm.at[wid], pid_v)
        pltpu.sync_copy(half_hbm.at[wid], half_v)

        lanes = lax.iota(jnp.int32, 16)

        def fire_gather(j, pbuf, sem):
            pltpu.async_copy(table_hbm.at[pid_v.at[j]], pbuf, sem)

        def drain_gather(pbuf, sem):
            pltpu.make_async_copy(
                table_hbm.at[pl.ds(0, CH)], pbuf, sem).wait()

        def extract(j, pbuf, obuf):
            for rc in range(CH // 16):
                hvec = half_v[j, pl.ds(rc * 16, 16)]
                rvec = lanes + rc * 16
                for c in range(D):
                    vals = plsc.load_gather(pbuf, [rvec, hvec + c])
                    plsc.store_scatter(
                        obuf, [rvec, jnp.full((16,), c, jnp.int32)], vals)

        def fire_store(j, obuf, sem):
            pltpu.async_copy(obuf, out_hbm.at[pl.ds(base + j * CH, CH)], sem)

        def drain_store(obuf, sem):
            pltpu.make_async_copy(
                obuf, out_hbm.at[pl.ds(base, CH)], sem).wait()

        fire_gather(0, pbuf_a, gsem_a)

        @pl.loop(0, n_chunks // 2)
        def pair(p):
            j0 = p * 2

            @pl.when(p > 0)
            def _():
                drain_store(obuf_b, ssem_b)
            fire_gather(j0 + 1, pbuf_b, gsem_b)
            drain_gather(pbuf_a, gsem_a)
            extract(j0, pbuf_a, obuf_a)
            fire_store(j0, obuf_a, ssem_a)
            drain_gather(pbuf_b, gsem_b)

            @pl.when(p < n_chunks // 2 - 1)
            def _():
                fire_gather(j0 + 2, pbuf_a, gsem_a)
            extract(j0 + 1, pbuf_b, obuf_b)
            fire_store(j0 + 1, obuf_b, ssem_b)
            drain_store(obuf_a, ssem_a)

        drain_store(obuf_b, ssem_b)

    return k(pid3, half3, table128)


def kernel(inputs, wte):
    n_rows = inputs.shape[0] * inputs.shape[1]
    n_chunks = n_rows // (NW * CH)
    perm = jnp.eye(2 * D, dtype=jnp.float32).reshape(2, D, 2 * D)
    table128 = jnp.tensordot(
        wte.reshape(wte.shape[0] // 2, 2, D), perm, axes=[[1, 2], [0, 1]],
        precision=jax.lax.Precision.HIGHEST)
    idx = inputs.reshape(NW, n_chunks, CH)
    pid = idx >> 1
    half = (idx & 1) * D
    out = _gather(pid, half, table128, n_rows)
    return out.reshape(inputs.shape[0], inputs.shape[1], D)


# speedup vs baseline: 1.4488x; 1.4488x over previous
import functools

import jax
import jax.numpy as jnp
from jax import lax
from jax.experimental import pallas as pl
from jax.experimental.pallas import tpu as pltpu
from jax.experimental.pallas import tpu_sc as plsc

D = 64
CH = 128
NB = 5
GR = NB * CH
NC = 2
NS = 16
NW = NC * NS


@functools.partial(jax.jit, static_argnums=(2,))
def _gather(idx3, table, n_rows):
    n_per_w = n_rows // NW
    n_chunks = n_per_w // CH
    n_groups = n_chunks // NB
    n_pairs = n_groups // 2

    @functools.partial(
        pl.kernel,
        out_type=jax.ShapeDtypeStruct((n_rows, D), jnp.float32),
        mesh=plsc.VectorSubcoreMesh(core_axis_name="c", subcore_axis_name="s"),
        compiler_params=pltpu.CompilerParams(use_tc_tiling_on_sc=False),
        scratch_types=[
            pltpu.VMEM((n_chunks, CH), jnp.int32),
            pltpu.VMEM((GR, D), jnp.float32),
            pltpu.VMEM((GR, D), jnp.float32),
            pltpu.SemaphoreType.DMA,
            pltpu.SemaphoreType.DMA,
            pltpu.SemaphoreType.DMA,
            pltpu.SemaphoreType.DMA,
        ],
    )
    def k(idx_hbm, table_hbm, out_hbm, idx_v, buf_a, buf_b,
          gsem_a, gsem_b, ssem_a, ssem_b):
        wid = lax.axis_index("s") * NC + lax.axis_index("c")
        base = wid * n_per_w

        pltpu.sync_copy(idx_hbm.at[wid], idx_v)

        def fire_gathers(g, buf, sem):
            for b in range(NB):
                pltpu.async_copy(
                    table_hbm.at[idx_v.at[g * NB + b]],
                    buf.at[pl.ds(b * CH, CH)],
                    sem,
                )

        def drain_gathers(buf, sem):
            pltpu.make_async_copy(
                out_hbm.at[pl.ds(0, GR)], buf, sem
            ).wait()

        def fire_store(g, buf, sem):
            pltpu.async_copy(
                buf, out_hbm.at[pl.ds(base + g * GR, GR)], sem
            )

        def drain_store(buf, sem):
            pltpu.make_async_copy(
                buf, out_hbm.at[pl.ds(base, GR)], sem
            ).wait()

        fire_gathers(0, buf_a, gsem_a)

        @pl.loop(0, n_pairs)
        def pair(p):
            g0 = p * 2

            @pl.when(p > 0)
            def _():
                drain_store(buf_b, ssem_b)
            fire_gathers(g0 + 1, buf_b, gsem_b)
            drain_gathers(buf_a, gsem_a)
            fire_store(g0, buf_a, ssem_a)
            drain_gathers(buf_b, gsem_b)
            drain_store(buf_a, ssem_a)

            @pl.when(p < n_pairs - 1)
            def _():
                fire_gathers(g0 + 2, buf_a, gsem_a)
            fire_store(g0 + 1, buf_b, ssem_b)

        drain_store(buf_b, ssem_b)

    return k(idx3, table)


def kernel(inputs, wte):
    n_rows = inputs.shape[0] * inputs.shape[1]
    idx3 = inputs.reshape(NW, n_rows // (NW * CH), CH)
    out = _gather(idx3, wte, n_rows)
    return out.reshape(inputs.shape[0], inputs.shape[1], D)
